# Initial kernel scaffold; baseline (speedup 1.0000x reference)
import functools

import jax
import jax.numpy as jnp
from jax import lax
from jax.experimental import pallas as pl
from jax.experimental.pallas import tpu as pltpu
from jax.experimental.pallas import tpu_sc as plsc

CUTOFF = 5.0


def _mlp_body(a_ref, w1_ref, b1_ref, w2_ref, b2_ref, out_ref):
    a = a_ref[...]
    h = lax.dot_general(a, w1_ref[...], (((1,), (1,)), ((), ())),
                        preferred_element_type=jnp.float32) + b1_ref[...]
    h = h * jax.nn.sigmoid(h)
    out_ref[...] = lax.dot_general(h, w2_ref[...], (((1,), (1,)), ((), ())),
                                   preferred_element_type=jnp.float32) + b2_ref[...]


def _mlp(a2, W1, b1, W2, b2):
    M, F = a2.shape
    return pl.pallas_call(
        _mlp_body,
        out_shape=jax.ShapeDtypeStruct((M, F), jnp.float32),
    )(a2, W1, b1.reshape(1, F), W2, b2.reshape(1, F))


def _make_gather(E, F, A, NN_per_batch):
    info = plsc.get_sparse_core_info()
    NC, NS = info.num_cores, info.num_subcores
    NW = NC * NS
    per_w = E // NW
    CH = 128
    n_chunks = per_w // CH
    w_per_batch = NW // (E // NN_per_batch)

    mesh = plsc.VectorSubcoreMesh(core_axis_name="c", subcore_axis_name="s")

    @functools.partial(
        pl.kernel,
        mesh=mesh,
        out_type=jax.ShapeDtypeStruct((E, F), jnp.float32),
        scratch_types=[
            pltpu.VMEM((CH,), jnp.int32),
            pltpu.VMEM((CH,), jnp.int32),
            pltpu.VMEM((CH, F), jnp.float32),
            pltpu.VMEM((CH, F), jnp.float32),
            pltpu.SemaphoreType.DMA,
            pltpu.SemaphoreType.DMA,
        ],
    )
    def gather_k(table_hbm, idx_hbm, out_hbm,
                 idx_v0, idx_v1, rows_v0, rows_v1, sem0, sem1):
        wid = lax.axis_index("s") * NC + lax.axis_index("c")
        base = wid * per_w
        b_add = (wid // w_per_batch) * A

        idx_bufs = (idx_v0, idx_v1)
        row_bufs = (rows_v0, rows_v1)
        sems = (sem0, sem1)

        def load_and_fire(ci, slot):
            cbase = base + ci * CH
            idx_v, rows_v, sem = idx_bufs[slot], row_bufs[slot], sems[slot]
            pltpu.sync_copy(idx_hbm.at[pl.ds(cbase, CH)], idx_v)
            for k in range(CH // 16):
                sl = pl.ds(k * 16, 16)
                idx_v[sl] = idx_v[sl] + b_add
            pltpu.async_copy(table_hbm.at[idx_v], rows_v, sem)

        def drain(ci, slot):
            cbase = base + ci * CH
            rows_v, sem = row_bufs[slot], sems[slot]
            pltpu.make_async_copy(table_hbm.at[idx_bufs[slot]], rows_v, sem).wait()
            pltpu.sync_copy(rows_v, out_hbm.at[pl.ds(cbase, CH)])

        load_and_fire(0, 0)

        def body(ci, _):
            slot = lax.rem(ci, 2)

            @pl.when(ci + 1 < n_chunks)
            def _():
                lax.switch(1 - slot, [lambda: load_and_fire(ci + 1, 0),
                                      lambda: load_and_fire(ci + 1, 1)])

            lax.switch(slot, [lambda: drain(ci, 0), lambda: drain(ci, 1)])
            return 0

        lax.fori_loop(0, n_chunks, body, 0)

    return gather_k


def _fuse_body(p_ref, aj_ref, rbf_ref, d_ref, nm_ref, a_ref, am_ref,
               wr_ref, br_ref, aout_ref, pout_ref, *, RB, NN, F):
    rbfm = lax.dot_general(rbf_ref[...], wr_ref[...], (((1,), (1,)), ((), ())),
                           preferred_element_type=jnp.float32) + br_ref[...]
    d = d_ref[...]
    x = d * (1.0 / CUTOFF)
    x2 = x * x
    x4 = x2 * x2
    x9 = x4 * x4 * x
    f = 1.0 + x9 * (-55.0 + x * (99.0 - 45.0 * x))
    cut = jnp.where(d < CUTOFF, f, 0.0)
    scale = (cut * nm_ref[...]).reshape(RB * NN, 1)

    am = am_ref[...]
    ai = jnp.broadcast_to(am[:, None, :], (RB, NN, F)).reshape(RB * NN, F)
    msij = ai * aj_ref[...] * rbfm * scale
    pout_ref[...] = p_ref[...] + msij
    aout_ref[...] = a_ref[...] + msij.reshape(RB, NN, F).sum(axis=1)


def _fuse(p_f, aj_f, rbf_f, d_r, nm_r, a2, amsij, W_rbf, b_rbf, RB):
    M, F = a2.shape
    E, R = rbf_f.shape
    NN = E // M
    eb = RB * NN
    body = functools.partial(_fuse_body, RB=RB, NN=NN, F=F)
    return pl.pallas_call(
        body,
        grid=(M // RB,),
        in_specs=[
            pl.BlockSpec((eb, F), lambda i: (i, 0)),
            pl.BlockSpec((eb, F), lambda i: (i, 0)),
            pl.BlockSpec((eb, R), lambda i: (i, 0)),
            pl.BlockSpec((RB, NN), lambda i: (i, 0)),
            pl.BlockSpec((RB, NN), lambda i: (i, 0)),
            pl.BlockSpec((RB, F), lambda i: (i, 0)),
            pl.BlockSpec((RB, F), lambda i: (i, 0)),
            pl.BlockSpec((F, R), lambda i: (0, 0)),
            pl.BlockSpec((1, F), lambda i: (0, 0)),
        ],
        out_specs=[
            pl.BlockSpec((RB, F), lambda i: (i, 0)),
            pl.BlockSpec((eb, F), lambda i: (i, 0)),
        ],
        out_shape=[
            jax.ShapeDtypeStruct((M, F), jnp.float32),
            jax.ShapeDtypeStruct((E, F), jnp.float32),
        ],
    )(p_f, aj_f, rbf_f, d_r, nm_r, a2, amsij, W_rbf, b_rbf.reshape(1, F))


def kernel(a, p, rbf, D, N, NM, W_rbf, b_rbf, W1, b1, W2, b2):
    B, A, NN, F = p.shape
    R = rbf.shape[-1]
    M = B * A
    E = M * NN

    a2 = a.reshape(M, F)
    amsij = _mlp(a2, W1, b1, W2, b2)

    idx = N.reshape(E)
    aj = _make_gather(E, F, A, A * NN)(amsij, idx)

    a_out2, p_out2 = _fuse(
        p.reshape(E, F), aj, rbf.reshape(E, R),
        D.reshape(M, NN), NM.reshape(M, NN),
        a2, amsij, W_rbf, b_rbf, RB=64,
    )
    return a_out2.reshape(B, A, F), p_out2.reshape(B, A, NN, F)


# speedup vs baseline: 946.8434x; 946.8434x over previous
import functools

import jax
import jax.numpy as jnp
from jax import lax
from jax.experimental import pallas as pl
from jax.experimental.pallas import tpu as pltpu
from jax.experimental.pallas import tpu_sc as plsc

CUTOFF = 5.0


def _mlp_body(a_ref, w1_ref, b1_ref, w2_ref, b2_ref, out_ref):
    a = a_ref[...]
    h = lax.dot_general(a, w1_ref[...], (((1,), (1,)), ((), ())),
                        preferred_element_type=jnp.float32) + b1_ref[...]
    h = h * jax.nn.sigmoid(h)
    out_ref[...] = lax.dot_general(h, w2_ref[...], (((1,), (1,)), ((), ())),
                                   preferred_element_type=jnp.float32) + b2_ref[...]


def _mlp(a2, W1, b1, W2, b2):
    M, F = a2.shape
    return pl.pallas_call(
        _mlp_body,
        out_shape=jax.ShapeDtypeStruct((M, F), jnp.float32),
    )(a2, W1, b1.reshape(1, F), W2, b2.reshape(1, F))


def _make_gather(E, F, A, NN_per_batch):
    info = plsc.get_sparse_core_info()
    NC, NS = info.num_cores, info.num_subcores
    NW = NC * NS
    per_w = E // NW
    CH = 128
    n_chunks = per_w // CH
    w_per_batch = NW // (E // NN_per_batch)

    mesh = plsc.VectorSubcoreMesh(core_axis_name="c", subcore_axis_name="s")

    @functools.partial(
        pl.kernel,
        mesh=mesh,
        out_type=jax.ShapeDtypeStruct((E, F), jnp.float32),
        scratch_types=[
            pltpu.VMEM((CH,), jnp.int32),
            pltpu.VMEM((CH,), jnp.int32),
            pltpu.VMEM((CH, F), jnp.float32),
            pltpu.VMEM((CH, F), jnp.float32),
            pltpu.SemaphoreType.DMA,
            pltpu.SemaphoreType.DMA,
        ],
    )
    def gather_k(table_hbm, idx_hbm, out_hbm,
                 idx_v0, idx_v1, rows_v0, rows_v1, sem0, sem1):
        wid = lax.axis_index("s") * NC + lax.axis_index("c")
        base = wid * per_w
        b_add = (wid // w_per_batch) * A

        idx_bufs = (idx_v0, idx_v1)
        row_bufs = (rows_v0, rows_v1)
        sems = (sem0, sem1)

        def load_and_fire(ci, slot):
            cbase = base + ci * CH
            idx_v, rows_v, sem = idx_bufs[slot], row_bufs[slot], sems[slot]
            pltpu.sync_copy(idx_hbm.at[pl.ds(cbase, CH)], idx_v)
            for k in range(CH // 16):
                sl = pl.ds(k * 16, 16)
                idx_v[sl] = idx_v[sl] + b_add
            pltpu.async_copy(table_hbm.at[idx_v], rows_v, sem)

        def drain(ci, slot):
            cbase = base + ci * CH
            rows_v, sem = row_bufs[slot], sems[slot]
            pltpu.make_async_copy(table_hbm.at[idx_bufs[slot]], rows_v, sem).wait()
            pltpu.sync_copy(rows_v, out_hbm.at[pl.ds(cbase, CH)])

        load_and_fire(0, 0)

        def body(ci, _):
            slot = lax.rem(ci, 2)

            @pl.when(ci + 1 < n_chunks)
            def _():
                lax.switch(1 - slot, [lambda: load_and_fire(ci + 1, 0),
                                      lambda: load_and_fire(ci + 1, 1)])

            lax.switch(slot, [lambda: drain(ci, 0), lambda: drain(ci, 1)])
            return 0

        lax.fori_loop(0, n_chunks, body, 0)

    return gather_k


def _fuse_body(p_ref, aj_ref, rbf_ref, d_ref, nm_ref, a_ref, am_ref,
               wr_ref, br_ref, aout_ref, pout_ref, *, RB, NN, F):
    rbfm = lax.dot_general(rbf_ref[...], wr_ref[...], (((1,), (1,)), ((), ())),
                           preferred_element_type=jnp.float32) + br_ref[...]
    d = d_ref[...]
    x = d * (1.0 / CUTOFF)
    x2 = x * x
    x4 = x2 * x2
    x9 = x4 * x4 * x
    f = 1.0 + x9 * (-55.0 + x * (99.0 - 45.0 * x))
    cut = jnp.where(d < CUTOFF, f, 0.0)
    scale = cut * nm_ref[...]

    rep = jnp.broadcast_to(scale[:, None, :], (RB, NN, NN)).reshape(RB * NN, NN)
    j_lane = lax.broadcasted_iota(jnp.int32, (RB * NN, NN), 1)
    j_row = lax.broadcasted_iota(jnp.int32, (RB * NN, NN), 0) % NN
    masked = jnp.where(j_lane == j_row, rep, 0.0)
    scale_e = lax.dot_general(masked, jnp.ones((NN, F), jnp.float32),
                              (((1,), (0,)), ((), ())),
                              preferred_element_type=jnp.float32)

    am = am_ref[...]
    ai = jnp.broadcast_to(am[:, None, :], (RB, NN, F)).reshape(RB * NN, F)
    msij = ai * aj_ref[...] * rbfm * scale_e
    pout_ref[...] = p_ref[...] + msij
    aout_ref[...] = a_ref[...] + msij.reshape(RB, NN, F).sum(axis=1)


def _fuse(p_f, aj_f, rbf_f, d_r, nm_r, a2, amsij, W_rbf, b_rbf, RB):
    M, F = a2.shape
    E, R = rbf_f.shape
    NN = E // M
    eb = RB * NN
    body = functools.partial(_fuse_body, RB=RB, NN=NN, F=F)
    return pl.pallas_call(
        body,
        grid=(M // RB,),
        in_specs=[
            pl.BlockSpec((eb, F), lambda i: (i, 0)),
            pl.BlockSpec((eb, F), lambda i: (i, 0)),
            pl.BlockSpec((eb, R), lambda i: (i, 0)),
            pl.BlockSpec((RB, NN), lambda i: (i, 0)),
            pl.BlockSpec((RB, NN), lambda i: (i, 0)),
            pl.BlockSpec((RB, F), lambda i: (i, 0)),
            pl.BlockSpec((RB, F), lambda i: (i, 0)),
            pl.BlockSpec((F, R), lambda i: (0, 0)),
            pl.BlockSpec((1, F), lambda i: (0, 0)),
        ],
        out_specs=[
            pl.BlockSpec((RB, F), lambda i: (i, 0)),
            pl.BlockSpec((eb, F), lambda i: (i, 0)),
        ],
        out_shape=[
            jax.ShapeDtypeStruct((M, F), jnp.float32),
            jax.ShapeDtypeStruct((E, F), jnp.float32),
        ],
    )(p_f, aj_f, rbf_f, d_r, nm_r, a2, amsij, W_rbf, b_rbf.reshape(1, F))


def kernel(a, p, rbf, D, N, NM, W_rbf, b_rbf, W1, b1, W2, b2):
    B, A, NN, F = p.shape
    R = rbf.shape[-1]
    M = B * A
    E = M * NN

    a2 = a.reshape(M, F)
    amsij = _mlp(a2, W1, b1, W2, b2)

    idx = N.reshape(E)
    aj = _make_gather(E, F, A, A * NN)(amsij, idx)

    a_out2, p_out2 = _fuse(
        p.reshape(E, F), aj, rbf.reshape(E, R),
        D.reshape(M, NN), NM.reshape(M, NN),
        a2, amsij, W_rbf, b_rbf, RB=64,
    )
    return a_out2.reshape(B, A, F), p_out2.reshape(B, A, NN, F)


# speedup vs baseline: 950.8153x; 1.0042x over previous
import functools

import jax
import jax.numpy as jnp
from jax import lax
from jax.experimental import pallas as pl
from jax.experimental.pallas import tpu as pltpu
from jax.experimental.pallas import tpu_sc as plsc

CUTOFF = 5.0


def _mlp_body(a_ref, w1_ref, b1_ref, w2_ref, b2_ref, out_ref):
    a = a_ref[...]
    h = lax.dot_general(a, w1_ref[...], (((1,), (1,)), ((), ())),
                        preferred_element_type=jnp.float32) + b1_ref[...]
    h = h * jax.nn.sigmoid(h)
    out_ref[...] = lax.dot_general(h, w2_ref[...], (((1,), (1,)), ((), ())),
                                   preferred_element_type=jnp.float32) + b2_ref[...]


def _mlp(a2, W1, b1, W2, b2):
    M, F = a2.shape
    return pl.pallas_call(
        _mlp_body,
        out_shape=jax.ShapeDtypeStruct((M, F), jnp.float32),
    )(a2, W1, b1.reshape(1, F), W2, b2.reshape(1, F))


def _make_gather(B, A, NN, F):
    info = plsc.get_sparse_core_info()
    NC, NS = info.num_cores, info.num_subcores
    NW = NC * NS
    E = B * A * NN
    per_w = E // NW
    AT = per_w // NN
    chunks_per_w = NW
    CH = per_w // chunks_per_w
    at_per_chunk = CH // NN
    w_per_b = NW // B
    L = 16

    mesh = plsc.VectorSubcoreMesh(core_axis_name="c", subcore_axis_name="s")

    @functools.partial(
        pl.kernel,
        mesh=mesh,
        out_type=jax.ShapeDtypeStruct((E, F), jnp.float32),
        scratch_types=[
            pltpu.VMEM((CH,), jnp.int32),
            pltpu.VMEM((CH,), jnp.int32),
            pltpu.VMEM((CH, F), jnp.float32),
            pltpu.VMEM((CH, F), jnp.float32),
            pltpu.SemaphoreType.DMA,
            pltpu.SemaphoreType.DMA,
        ],
    )
    def gather_k(table_hbm, idx_hbm, out_hbm,
                 idx_v0, idx_v1, rows_v0, rows_v1, sem0, sem1):
        wid = lax.axis_index("s") * NC + lax.axis_index("c")
        base = wid * per_w
        b_add = (wid // w_per_b) * A

        idx_bufs = (idx_v0, idx_v1)
        row_bufs = (rows_v0, rows_v1)
        sems = (sem0, sem1)

        def load_and_fire(ci, slot):
            cbase = base + ci * CH
            idx_v, rows_v, sem = idx_bufs[slot], row_bufs[slot], sems[slot]
            pltpu.sync_copy(idx_hbm.at[pl.ds(cbase, CH)], idx_v)
            for m in range(CH // L):
                sl = pl.ds(m * L, L)
                idx_v[sl] = idx_v[sl] + b_add
            pltpu.async_copy(table_hbm.at[idx_v], rows_v, sem)

        def drain(ci, slot):
            cbase = base + ci * CH
            rows_v, sem = row_bufs[slot], sems[slot]
            pltpu.make_async_copy(table_hbm.at[idx_bufs[slot]], rows_v, sem).wait()
            pltpu.sync_copy(rows_v, out_hbm.at[pl.ds(cbase, CH)])

        load_and_fire(0, 0)

        def body(ci, _):
            slot = lax.rem(ci, 2)

            @pl.when(ci + 1 < chunks_per_w)
            def _():
                lax.switch(1 - slot, [lambda: load_and_fire(ci + 1, 0),
                                      lambda: load_and_fire(ci + 1, 1)])

            lax.switch(slot, [lambda: drain(ci, 0), lambda: drain(ci, 1)])
            return 0

        lax.fori_loop(0, chunks_per_w, body, 0)

    return gather_k


def _fuse_body(p_ref, aj_ref, rbf_ref, d_ref, nm_ref, a_ref, am_ref,
               wr_ref, br_ref, aout_ref, pout_ref, *, RB, NN, F):
    rbfm = lax.dot_general(rbf_ref[...], wr_ref[...], (((1,), (0,)), ((), ())),
                           preferred_element_type=jnp.float32) + br_ref[...]
    d = d_ref[0]
    x = d * (1.0 / CUTOFF)
    x2 = x * x
    x4 = x2 * x2
    x9 = x4 * x4 * x
    f = 1.0 + x9 * (-55.0 + x * (99.0 - 45.0 * x))
    cut = jnp.where(d < CUTOFF, f, 0.0)
    scale = jnp.swapaxes(cut * nm_ref[0], 0, 1)

    rep = jnp.broadcast_to(scale[:, None, :], (RB, NN, NN)).reshape(RB * NN, NN)
    j_lane = lax.broadcasted_iota(jnp.int32, (RB * NN, NN), 1)
    j_row = lax.broadcasted_iota(jnp.int32, (RB * NN, NN), 0) % NN
    masked = jnp.where(j_lane == j_row, rep, 0.0)
    scale_e = lax.dot_general(masked, jnp.ones((NN, F), jnp.float32),
                              (((1,), (0,)), ((), ())),
                              preferred_element_type=jnp.float32)

    am = am_ref[...]
    ai = jnp.broadcast_to(am[:, None, :], (RB, NN, F)).reshape(RB * NN, F)
    msij = ai * aj_ref[...] * rbfm * scale_e
    pout_ref[...] = p_ref[...] + msij
    aout_ref[...] = a_ref[...] + msij.reshape(RB, NN, F).sum(axis=1)


def _fuse(p_f, aj_f, rbf_f, d_v, nm_v, a2, amsij, W_rbfT, b_rbf, RB):
    M, F = a2.shape
    E, R = rbf_f.shape
    NN = E // M
    B = d_v.shape[0]
    steps_per_b = M // B // RB
    eb = RB * NN
    body = functools.partial(_fuse_body, RB=RB, NN=NN, F=F)
    return pl.pallas_call(
        body,
        grid=(M // RB,),
        in_specs=[
            pl.BlockSpec((eb, F), lambda g: (g, 0)),
            pl.BlockSpec((eb, F), lambda g: (g, 0)),
            pl.BlockSpec((eb, R), lambda g: (g, 0)),
            pl.BlockSpec((1, NN, RB),
                         lambda g: (g // steps_per_b, 0, g % steps_per_b)),
            pl.BlockSpec((1, NN, RB),
                         lambda g: (g // steps_per_b, 0, g % steps_per_b)),
            pl.BlockSpec((RB, F), lambda g: (g, 0)),
            pl.BlockSpec((RB, F), lambda g: (g, 0)),
            pl.BlockSpec((R, F), lambda g: (0, 0)),
            pl.BlockSpec((1, F), lambda g: (0, 0)),
        ],
        out_specs=[
            pl.BlockSpec((RB, F), lambda g: (g, 0)),
            pl.BlockSpec((eb, F), lambda g: (g, 0)),
        ],
        out_shape=[
            jax.ShapeDtypeStruct((M, F), jnp.float32),
            jax.ShapeDtypeStruct((E, F), jnp.float32),
        ],
    )(p_f, aj_f, rbf_f, d_v, nm_v, a2, amsij, W_rbfT, b_rbf.reshape(1, F))


def kernel(a, p, rbf, D, N, NM, W_rbf, b_rbf, W1, b1, W2, b2):
    B, A, NN, F = p.shape
    R = rbf.shape[-1]
    M = B * A
    E = M * NN

    a2 = a.reshape(M, F)
    amsij = _mlp(a2, W1, b1, W2, b2)

    d_v = D.transpose(0, 2, 1)
    nm_v = NM.transpose(0, 2, 1)
    w_rbf_t = W_rbf.transpose(1, 0)

    aj = _make_gather(B, A, NN, F)(amsij, N.reshape(E))

    a_out2, p_out2 = _fuse(
        p.reshape(E, F), aj, rbf.reshape(E, R),
        d_v, nm_v, a2, amsij, w_rbf_t, b_rbf, RB=128,
    )
    return a_out2.reshape(B, A, F), p_out2.reshape(B, A, NN, F)
